# baseline (speedup 1.0000x reference)
import functools

import jax
import jax.numpy as jnp
from jax import lax
from jax.experimental import pallas as pl
from jax.experimental.pallas import tpu as pltpu
from jax.experimental.pallas import tpu_sc as plsc

_N_CAT = 26
_NUM_CATS = 1_000_000
_BATCH = 16384
_CAT0 = 13
_TOT = _BATCH * _N_CAT
_NC, _NS = 2, 16
_NW = _NC * _NS
_PER_W = _TOT // _NW
_CHUNK = 128
_NCH = _PER_W // _CHUNK
_FIRE = 8
_NLOOP = _NCH // _FIRE


def _sc_gather(table, idx3):
    mesh = plsc.VectorSubcoreMesh(core_axis_name="c", subcore_axis_name="s")

    @functools.partial(
        pl.kernel,
        out_type=jax.ShapeDtypeStruct((_NW, _NCH, _CHUNK), jnp.float32),
        mesh=mesh,
        scratch_types=[
            pltpu.VMEM((_NCH, _CHUNK), jnp.int32),
            pltpu.VMEM((_NCH, _CHUNK), jnp.float32),
            pltpu.SemaphoreType.DMA,
        ],
    )
    def k(table_hbm, idx_hbm, out_hbm, idx_v, dst_v, sem):
        wid = lax.axis_index("s") * _NC + lax.axis_index("c")
        pltpu.sync_copy(idx_hbm.at[wid], idx_v)

        def body(o, carry):
            base = o * _FIRE
            descs = [
                pltpu.async_copy(
                    table_hbm.at[idx_v.at[base + j]], dst_v.at[base + j], sem
                )
                for j in range(_FIRE)
            ]
            for d in descs:
                d.wait()
            return carry

        lax.fori_loop(0, _NLOOP, body, 0)
        pltpu.sync_copy(dst_v, out_hbm.at[wid])

    return k(table, idx3)


def kernel(inputs, embeddings):
    table = embeddings.reshape(-1)
    offs = jnp.arange(_N_CAT, dtype=jnp.int32) * _NUM_CATS
    idx = inputs[:, _CAT0:].astype(jnp.int32) + offs[None, :]
    gathered = _sc_gather(table, idx.reshape(_NW, _NCH, _CHUNK))
    return jnp.concatenate(
        [inputs[:, :_CAT0], gathered.reshape(_BATCH, _N_CAT)], axis=1
    )
